# baseline (speedup 1.0000x reference)
import functools
import math

import jax
import jax.numpy as jnp
from jax import lax
from jax.experimental import pallas as pl
from jax.experimental.pallas import tpu as pltpu
from jax.experimental.pallas import tpu_sc as plsc

B, T, D = 2, 2048, 1024
E = 8
DFF = 4 * D
N = B * T
A = 2 * N
BM = 256
NT = 40
NROWS = NT * BM
EP = 128
NEG = -1e30

NC, NS = 2, 16
CS = A // NS


def _router_body(x_ref, wg_ref, i1_ref, i2_ref, w1_ref, w2_ref):
    x = x_ref[...]
    logits = lax.dot_general(x, wg_ref[...], (((1,), (0,)), ((), ())),
                             preferred_element_type=jnp.float32)
    lane = lax.broadcasted_iota(jnp.int32, logits.shape, 1)
    valid = lane < E
    logits = jnp.where(valid, logits, NEG)
    m1 = jnp.max(logits, axis=1, keepdims=True)
    i1 = jnp.min(jnp.where(logits == m1, lane, EP), axis=1, keepdims=True)
    l2 = jnp.where(lane == i1, NEG, logits)
    m2 = jnp.max(l2, axis=1, keepdims=True)
    i2 = jnp.min(jnp.where(l2 == m2, lane, EP), axis=1, keepdims=True)
    e = jnp.exp(m2 - m1)
    w0 = 1.0 / (1.0 + e)
    i1_ref[...] = i1[:, 0]
    i2_ref[...] = i2[:, 0]
    w1_ref[...] = w0[:, 0]
    w2_ref[...] = (e * w0)[:, 0]


def _meta_body(i1_hbm, i2_hbm, pos_hbm, te_hbm, ids_v, pos_v, te_v, sem):
    c = lax.axis_index("c")
    s = lax.axis_index("s")
    lane = lax.iota(jnp.int32, 16)

    pltpu.sync_copy(i1_hbm, ids_v.at[pl.ds(0, N)])
    pltpu.sync_copy(i2_hbm, ids_v.at[pl.ds(N, N)])

    def h_step(v, hist):
        ev = ids_v[pl.ds(v * 16, 16)]
        for e in range(E):
            cnt = plsc.all_reduce_population_count(ev == e)
            hist = hist + jnp.where(lane == e, cnt, 0)
        return hist

    myv0 = s * (CS // 16)
    mybase = lax.fori_loop(0, myv0, h_step, jnp.zeros((16,), jnp.int32))
    g = lax.fori_loop(myv0, A // 16, h_step, mybase)
    ctiles = (g + (BM - 1)) // BM
    csz = ctiles * BM
    coff_incl = plsc.cumsum(csz)
    padded_off = coff_incl - csz
    base_vec = padded_off + mybase

    bases0 = tuple(jnp.sum(jnp.where(lane == e, base_vec, 0)) for e in range(E))

    def r_step(v, bases):
        ev = ids_v[pl.ds((myv0 + v) * 16, 16)]
        posv = jnp.zeros((16,), jnp.int32)
        new_bases = []
        for e in range(E):
            msk = ev == e
            r = plsc.cumsum(jnp.where(msk, 1, 0))
            posv = jnp.where(msk, bases[e] + r - 1, posv)
            new_bases.append(bases[e] + jnp.sum(jnp.where(msk, 1, 0)))
        pos_v[pl.ds(v * 16, 16)] = posv
        return tuple(new_bases)

    lax.fori_loop(0, CS // 16, r_step, bases0)

    wc = jnp.where(s >= 8, 1, 0)

    @pl.when(c == wc)
    def _():
        pltpu.sync_copy(pos_v, pos_hbm.at[pl.ds(s * CS, CS)])

    btile = plsc.cumsum(ctiles)
    bts = tuple(jnp.sum(jnp.where(lane == e, btile, 0)) for e in range(E))
    for vi in range(3):
        tvec = lax.iota(jnp.int32, 16) + vi * 16
        tev = jnp.zeros((16,), jnp.int32)
        for e in range(E):
            tev = tev + jnp.where(tvec >= bts[e], 1, 0)
        tev = jnp.minimum(tev, E - 1)
        if vi == 2:
            tev = jnp.where(tvec == 47, bts[E - 1], tev)
        te_v[pl.ds(vi * 16, 16)] = tev

    @pl.when((c == 0) & (s == 0))
    def _():
        pltpu.sync_copy(te_v, te_hbm)


GQ = 8
GR = 32


def _gather_body(x_hbm, pos_hbm, xs_hbm, idx_v, b0, b1, sem0, sem1):
    c = lax.axis_index("c")
    s = lax.axis_index("s")
    wid = s * NC + c
    row_base = lax.rem(wid, NS) * (GQ * GR)
    pltpu.sync_copy(pos_hbm.at[wid], idx_v)

    copies = []
    for q in range(GQ):
        b = b0 if q % 2 == 0 else b1
        sem = sem0 if q % 2 == 0 else sem1
        if q >= 2:
            copies[q - 2].wait()
        pltpu.sync_copy(x_hbm.at[pl.ds(row_base + q * GR, GR)], b)
        copies.append(pltpu.async_copy(b, xs_hbm.at[idx_v.at[q]], sem))
    copies[GQ - 2].wait()
    copies[GQ - 1].wait()


MLP_BK = 512


def _mlp_body(te_ref, xs_ref, w1_hbm, w2_hbm, out_ref,
              w1b, w2b, ordr, s1a, s2a, s1b, s2b):
    m = pl.program_id(0)
    e = te_ref[m]
    sems = ((s1a, s2a), (s1b, s2b))

    def fetch(slot, eidx, sp):
        pltpu.make_async_copy(w1_hbm.at[eidx], w1b.at[slot], sp[0]).start()
        pltpu.make_async_copy(w2_hbm.at[eidx], w2b.at[slot], sp[1]).start()

    def wait_slot(slot, eidx, sp):
        pltpu.make_async_copy(w1_hbm.at[eidx], w1b.at[slot], sp[0]).wait()
        pltpu.make_async_copy(w2_hbm.at[eidx], w2b.at[slot], sp[1]).wait()

    nu = te_ref[47]
    active = m < nu
    prev_e = te_ref[jnp.maximum(m - 1, 0)]
    boundary = jnp.logical_and((m == 0) | (e != prev_e), active)

    @pl.when(m == 0)
    def _():
        ordr[0] = 0
        fetch(0, e, sems[0])

    @pl.when(boundary)
    def _():
        @pl.when(m > 0)
        def _():
            ordr[0] = ordr[0] + 1
        slot = lax.rem(ordr[0], 2)

        @pl.when(slot == 0)
        def _():
            wait_slot(0, e, sems[0])

        @pl.when(slot == 1)
        def _():
            wait_slot(1, e, sems[1])

        def scan(i, st):
            found, val = st
            cand = te_ref[i]
            take = jnp.logical_and(
                jnp.logical_not(found),
                jnp.logical_and(jnp.logical_and(i > m, i < nu), cand != e))
            return (jnp.logical_or(found, take), jnp.where(take, cand, val))

        _, en = lax.fori_loop(0, NT, scan, (False, jnp.int32(-1)))
        nslot = lax.rem(ordr[0] + 1, 2)

        @pl.when(jnp.logical_and(en >= 0, nslot == 0))
        def _():
            fetch(0, en, sems[0])

        @pl.when(jnp.logical_and(en >= 0, nslot == 1))
        def _():
            fetch(1, en, sems[1])

    @pl.when(active)
    def _():
        slot = lax.rem(ordr[0], 2)
        xb = xs_ref[...].astype(jnp.bfloat16)
        acc = jnp.zeros((BM, D), jnp.float32)
        for j in range(DFF // MLP_BK):
            w1j = w1b[slot, :, pl.ds(j * MLP_BK, MLP_BK)]
            hj = lax.dot_general(xb, w1j, (((1,), (0,)), ((), ())),
                                 preferred_element_type=jnp.float32)
            hj = 0.5 * hj * (1.0 + lax.erf(hj * (1.0 / math.sqrt(2.0))))
            w2j = w2b[slot, pl.ds(j * MLP_BK, MLP_BK), :].astype(jnp.bfloat16)
            acc = acc + lax.dot_general(hj.astype(jnp.bfloat16), w2j,
                                        (((1,), (0,)), ((), ())),
                                        preferred_element_type=jnp.float32)
        out_ref[...] = acc


CQ = 8
CT = 16


def _combine_body(ys_hbm, posA_hbm, posB_hbm, w1_hbm, w2_hbm, out_hbm,
                  idx0_v, idx1_v, w0_v, w1v_v, r0a, r1a, r0b, r1b, ob,
                  semg0, semg1, semg2, semg3):
    c = lax.axis_index("c")
    s = lax.axis_index("s")
    wid = s * NC + c
    n0 = wid * (CQ * CT)
    lane = lax.iota(jnp.int32, 16)
    pltpu.sync_copy(posA_hbm.at[wid], idx0_v)
    pltpu.sync_copy(posB_hbm.at[wid], idx1_v)
    pltpu.sync_copy(w1_hbm.at[pl.ds(n0, CQ * CT)], w0_v)
    pltpu.sync_copy(w2_hbm.at[pl.ds(n0, CQ * CT)], w1v_v)

    bufs = ((r0a, r1a, semg0, semg1), (r0b, r1b, semg2, semg3))

    def issue(q):
        b0, b1, sg0, sg1 = bufs[q % 2]
        cp0 = pltpu.async_copy(ys_hbm.at[idx0_v.at[q]], b0, sg0)
        cp1 = pltpu.async_copy(ys_hbm.at[idx1_v.at[q]], b1, sg1)
        return cp0, cp1

    cps = [issue(0)]
    for q in range(CQ):
        if q + 1 < CQ:
            cps.append(issue(q + 1))
        cps[q][0].wait()
        cps[q][1].wait()
        b0, b1 = bufs[q % 2][0], bufs[q % 2][1]
        wa = w0_v[pl.ds(q * 16, 16)]
        wb = w1v_v[pl.ds(q * 16, 16)]

        def t_step(t, _):
            w0s = jnp.sum(jnp.where(lane == t, wa, 0.0))
            w1s = jnp.sum(jnp.where(lane == t, wb, 0.0))
            for j in range(D // 16):
                ob[t, pl.ds(j * 16, 16)] = (
                    w0s * b0[t, pl.ds(j * 16, 16)]
                    + w1s * b1[t, pl.ds(j * 16, 16)])
            return 0

        lax.fori_loop(0, CT, t_step, 0)
        pltpu.sync_copy(ob, out_hbm.at[pl.ds(n0 + q * CT, CT)])


@jax.jit
def kernel(x, Wg, W1, W2):
    W1b = W1.astype(jnp.bfloat16)
    x2 = x.reshape(N, D)
    wgp = jnp.pad(Wg, ((0, 0), (0, EP - E)))

    BR = 512
    i1, i2, w1r, w2r = pl.pallas_call(
        _router_body,
        grid=(N // BR,),
        in_specs=[
            pl.BlockSpec((BR, D), lambda i: (i, 0)),
            pl.BlockSpec((D, EP), lambda i: (0, 0)),
        ],
        out_specs=[
            pl.BlockSpec((BR,), lambda i: (i,)),
            pl.BlockSpec((BR,), lambda i: (i,)),
            pl.BlockSpec((BR,), lambda i: (i,)),
            pl.BlockSpec((BR,), lambda i: (i,)),
        ],
        out_shape=[
            jax.ShapeDtypeStruct((N,), jnp.int32),
            jax.ShapeDtypeStruct((N,), jnp.int32),
            jax.ShapeDtypeStruct((N,), jnp.float32),
            jax.ShapeDtypeStruct((N,), jnp.float32),
        ],
    )(x2, wgp)

    mesh = plsc.VectorSubcoreMesh(core_axis_name="c", subcore_axis_name="s")

    meta = pl.kernel(
        _meta_body,
        out_type=[
            jax.ShapeDtypeStruct((A,), jnp.int32),
            jax.ShapeDtypeStruct((48,), jnp.int32),
        ],
        mesh=mesh,
        compiler_params=pltpu.CompilerParams(needs_layout_passes=False),
        scratch_types=[
            pltpu.VMEM((A,), jnp.int32),
            pltpu.VMEM((CS,), jnp.int32),
            pltpu.VMEM((48,), jnp.int32),
            pltpu.SemaphoreType.DMA,
        ],
    )
    pos, te = meta(i1, i2)

    pos3 = pos.reshape(NC * NS, GQ, GR)
    xs = pl.kernel(
        _gather_body,
        out_type=jax.ShapeDtypeStruct((NROWS, D), jnp.float32),
        mesh=mesh,
        compiler_params=pltpu.CompilerParams(needs_layout_passes=False),
        scratch_types=[
            pltpu.VMEM((GQ, GR), jnp.int32),
            pltpu.VMEM((GR, D), jnp.float32),
            pltpu.VMEM((GR, D), jnp.float32),
            pltpu.SemaphoreType.DMA,
            pltpu.SemaphoreType.DMA,
        ],
    )(x2, pos3)

    grid_spec = pltpu.PrefetchScalarGridSpec(
        num_scalar_prefetch=1,
        grid=(NT,),
        in_specs=[
            pl.BlockSpec((BM, D), lambda m, te_r: (m, 0)),
            pl.BlockSpec(memory_space=pl.ANY),
            pl.BlockSpec(memory_space=pl.ANY),
        ],
        out_specs=pl.BlockSpec((BM, D), lambda m, te_r: (m, 0)),
        scratch_shapes=[
            pltpu.VMEM((2, D, DFF), jnp.bfloat16),
            pltpu.VMEM((2, DFF, D), jnp.float32),
            pltpu.SMEM((2,), jnp.int32),
            pltpu.SemaphoreType.DMA,
            pltpu.SemaphoreType.DMA,
            pltpu.SemaphoreType.DMA,
            pltpu.SemaphoreType.DMA,
        ],
    )
    ys = pl.pallas_call(
        _mlp_body,
        grid_spec=grid_spec,
        out_shape=jax.ShapeDtypeStruct((NROWS, D), jnp.float32),
        compiler_params=pltpu.CompilerParams(vmem_limit_bytes=100 * 1024 * 1024),
    )(te, xs, W1b, W2)

    posA = pos[:N].reshape(NC * NS, CQ, CT)
    posB = pos[N:].reshape(NC * NS, CQ, CT)
    out = pl.kernel(
        _combine_body,
        out_type=jax.ShapeDtypeStruct((N, D), jnp.float32),
        mesh=mesh,
        compiler_params=pltpu.CompilerParams(needs_layout_passes=False),
        scratch_types=[
            pltpu.VMEM((CQ, CT), jnp.int32),
            pltpu.VMEM((CQ, CT), jnp.int32),
            pltpu.VMEM((CQ * CT,), jnp.float32),
            pltpu.VMEM((CQ * CT,), jnp.float32),
            pltpu.VMEM((CT, D), jnp.float32),
            pltpu.VMEM((CT, D), jnp.float32),
            pltpu.VMEM((CT, D), jnp.float32),
            pltpu.VMEM((CT, D), jnp.float32),
            pltpu.VMEM((CT, D), jnp.float32),
            pltpu.SemaphoreType.DMA,
            pltpu.SemaphoreType.DMA,
            pltpu.SemaphoreType.DMA,
            pltpu.SemaphoreType.DMA,
        ],
    )(ys, posA, posB, w1r, w2r)
    return out.reshape(B, T, D)


# speedup vs baseline: 1.0945x; 1.0945x over previous
import functools
import math

import jax
import jax.numpy as jnp
from jax import lax
from jax.experimental import pallas as pl
from jax.experimental.pallas import tpu as pltpu
from jax.experimental.pallas import tpu_sc as plsc

B, T, D = 2, 2048, 1024
E = 8
DFF = 4 * D
N = B * T
A = 2 * N
BM = 256
NT = 40
NROWS = NT * BM
EP = 128
NEG = -1e30

NC, NS = 2, 16
CS = A // NS


def _router_body(x_ref, wg_ref, i1_ref, i2_ref, w1_ref, w2_ref):
    x = x_ref[...]
    logits = lax.dot_general(x, wg_ref[...], (((1,), (0,)), ((), ())),
                             preferred_element_type=jnp.float32)
    lane = lax.broadcasted_iota(jnp.int32, logits.shape, 1)
    valid = lane < E
    logits = jnp.where(valid, logits, NEG)
    m1 = jnp.max(logits, axis=1, keepdims=True)
    i1 = jnp.min(jnp.where(logits == m1, lane, EP), axis=1, keepdims=True)
    l2 = jnp.where(lane == i1, NEG, logits)
    m2 = jnp.max(l2, axis=1, keepdims=True)
    i2 = jnp.min(jnp.where(l2 == m2, lane, EP), axis=1, keepdims=True)
    e = jnp.exp(m2 - m1)
    w0 = 1.0 / (1.0 + e)
    i1_ref[...] = i1[:, 0]
    i2_ref[...] = i2[:, 0]
    w1_ref[...] = w0[:, 0]
    w2_ref[...] = (e * w0)[:, 0]


def _meta_body(i1_hbm, i2_hbm, pos_hbm, te_hbm, ids_v, pos_v, te_v, sem):
    c = lax.axis_index("c")
    s = lax.axis_index("s")
    lane = lax.iota(jnp.int32, 16)

    pltpu.sync_copy(i1_hbm, ids_v.at[pl.ds(0, N)])
    pltpu.sync_copy(i2_hbm, ids_v.at[pl.ds(N, N)])

    def h_step(v, hist):
        ev = ids_v[pl.ds(v * 16, 16)]
        for e in range(E):
            cnt = plsc.all_reduce_population_count(ev == e)
            hist = hist + jnp.where(lane == e, cnt, 0)
        return hist

    myv0 = s * (CS // 16)
    mybase = lax.fori_loop(0, myv0, h_step, jnp.zeros((16,), jnp.int32))
    g = lax.fori_loop(myv0, A // 16, h_step, mybase)
    ctiles = (g + (BM - 1)) // BM
    csz = ctiles * BM
    coff_incl = plsc.cumsum(csz)
    padded_off = coff_incl - csz
    base_vec = padded_off + mybase

    bases0 = tuple(jnp.sum(jnp.where(lane == e, base_vec, 0)) for e in range(E))

    def r_step(v, bases):
        ev = ids_v[pl.ds((myv0 + v) * 16, 16)]
        posv = jnp.zeros((16,), jnp.int32)
        new_bases = []
        for e in range(E):
            msk = ev == e
            r = plsc.cumsum(jnp.where(msk, 1, 0))
            posv = jnp.where(msk, bases[e] + r - 1, posv)
            new_bases.append(bases[e] + jnp.sum(jnp.where(msk, 1, 0)))
        pos_v[pl.ds(v * 16, 16)] = posv
        return tuple(new_bases)

    lax.fori_loop(0, CS // 16, r_step, bases0)

    wc = jnp.where(s >= 8, 1, 0)

    @pl.when(c == wc)
    def _():
        pltpu.sync_copy(pos_v, pos_hbm.at[pl.ds(s * CS, CS)])

    btile = plsc.cumsum(ctiles)
    bts = tuple(jnp.sum(jnp.where(lane == e, btile, 0)) for e in range(E))
    for vi in range(3):
        tvec = lax.iota(jnp.int32, 16) + vi * 16
        tev = jnp.zeros((16,), jnp.int32)
        for e in range(E):
            tev = tev + jnp.where(tvec >= bts[e], 1, 0)
        tev = jnp.minimum(tev, E - 1)
        if vi == 2:
            tev = jnp.where(tvec == 47, bts[E - 1], tev)
        te_v[pl.ds(vi * 16, 16)] = tev

    @pl.when((c == 0) & (s == 0))
    def _():
        pltpu.sync_copy(te_v, te_hbm)


GQ = 8
GR = 32


def _gather_body(x_hbm, pos_hbm, xs_hbm, idx_v, b0, b1, sem0, sem1):
    c = lax.axis_index("c")
    s = lax.axis_index("s")
    wid = s * NC + c
    row_base = lax.rem(wid, NS) * (GQ * GR)
    pltpu.sync_copy(pos_hbm.at[wid], idx_v)

    copies = []
    for q in range(GQ):
        b = b0 if q % 2 == 0 else b1
        sem = sem0 if q % 2 == 0 else sem1
        if q >= 2:
            copies[q - 2].wait()
        pltpu.sync_copy(x_hbm.at[pl.ds(row_base + q * GR, GR)], b)
        copies.append(pltpu.async_copy(b, xs_hbm.at[idx_v.at[q]], sem))
    copies[GQ - 2].wait()
    copies[GQ - 1].wait()


MLP_BK = 2048


def _mlp_body(te_ref, xs_ref, w1_hbm, w2_hbm, out_ref,
              w1b, w2b, ordr, s1a, s2a, s1b, s2b):
    m = pl.program_id(0)
    e = te_ref[m]
    sems = ((s1a, s2a), (s1b, s2b))

    def fetch(slot, eidx, sp):
        pltpu.make_async_copy(w1_hbm.at[eidx], w1b.at[slot], sp[0]).start()
        pltpu.make_async_copy(w2_hbm.at[eidx], w2b.at[slot], sp[1]).start()

    def wait_slot(slot, eidx, sp):
        pltpu.make_async_copy(w1_hbm.at[eidx], w1b.at[slot], sp[0]).wait()
        pltpu.make_async_copy(w2_hbm.at[eidx], w2b.at[slot], sp[1]).wait()

    nu = te_ref[47]
    active = m < nu
    prev_e = te_ref[jnp.maximum(m - 1, 0)]
    boundary = jnp.logical_and((m == 0) | (e != prev_e), active)

    @pl.when(m == 0)
    def _():
        ordr[0] = 0
        fetch(0, e, sems[0])

    @pl.when(boundary)
    def _():
        @pl.when(m > 0)
        def _():
            ordr[0] = ordr[0] + 1
        slot = lax.rem(ordr[0], 2)

        @pl.when(slot == 0)
        def _():
            wait_slot(0, e, sems[0])

        @pl.when(slot == 1)
        def _():
            wait_slot(1, e, sems[1])

        def scan(i, st):
            found, val = st
            cand = te_ref[i]
            take = jnp.logical_and(
                jnp.logical_not(found),
                jnp.logical_and(jnp.logical_and(i > m, i < nu), cand != e))
            return (jnp.logical_or(found, take), jnp.where(take, cand, val))

        _, en = lax.fori_loop(0, NT, scan, (False, jnp.int32(-1)))
        nslot = lax.rem(ordr[0] + 1, 2)

        @pl.when(jnp.logical_and(en >= 0, nslot == 0))
        def _():
            fetch(0, en, sems[0])

        @pl.when(jnp.logical_and(en >= 0, nslot == 1))
        def _():
            fetch(1, en, sems[1])

    @pl.when(active)
    def _():
        slot = lax.rem(ordr[0], 2)
        xb = xs_ref[...].astype(jnp.bfloat16)
        acc = jnp.zeros((BM, D), jnp.float32)
        for j in range(DFF // MLP_BK):
            w1j = w1b[slot, :, pl.ds(j * MLP_BK, MLP_BK)]
            hj = lax.dot_general(xb, w1j, (((1,), (0,)), ((), ())),
                                 preferred_element_type=jnp.float32)
            hj = 0.5 * hj * (1.0 + lax.erf(hj * (1.0 / math.sqrt(2.0))))
            w2j = w2b[slot, pl.ds(j * MLP_BK, MLP_BK), :].astype(jnp.bfloat16)
            acc = acc + lax.dot_general(hj.astype(jnp.bfloat16), w2j,
                                        (((1,), (0,)), ((), ())),
                                        preferred_element_type=jnp.float32)
        out_ref[...] = acc


CQ = 8
CT = 16


def _combine_body(ys_hbm, posA_hbm, posB_hbm, w1_hbm, w2_hbm, out_hbm,
                  idx0_v, idx1_v, w0_v, w1v_v, r0a, r1a, r0b, r1b, ob,
                  semg0, semg1, semg2, semg3):
    c = lax.axis_index("c")
    s = lax.axis_index("s")
    wid = s * NC + c
    n0 = wid * (CQ * CT)
    lane = lax.iota(jnp.int32, 16)
    pltpu.sync_copy(posA_hbm.at[wid], idx0_v)
    pltpu.sync_copy(posB_hbm.at[wid], idx1_v)
    pltpu.sync_copy(w1_hbm.at[pl.ds(n0, CQ * CT)], w0_v)
    pltpu.sync_copy(w2_hbm.at[pl.ds(n0, CQ * CT)], w1v_v)

    bufs = ((r0a, r1a, semg0, semg1), (r0b, r1b, semg2, semg3))

    def issue(q):
        b0, b1, sg0, sg1 = bufs[q % 2]
        cp0 = pltpu.async_copy(ys_hbm.at[idx0_v.at[q]], b0, sg0)
        cp1 = pltpu.async_copy(ys_hbm.at[idx1_v.at[q]], b1, sg1)
        return cp0, cp1

    cps = [issue(0)]
    for q in range(CQ):
        if q + 1 < CQ:
            cps.append(issue(q + 1))
        cps[q][0].wait()
        cps[q][1].wait()
        b0, b1 = bufs[q % 2][0], bufs[q % 2][1]
        wa = w0_v[pl.ds(q * 16, 16)]
        wb = w1v_v[pl.ds(q * 16, 16)]

        def t_step(t, _):
            w0s = jnp.sum(jnp.where(lane == t, wa, 0.0))
            w1s = jnp.sum(jnp.where(lane == t, wb, 0.0))
            for j in range(D // 16):
                ob[t, pl.ds(j * 16, 16)] = (
                    w0s * b0[t, pl.ds(j * 16, 16)]
                    + w1s * b1[t, pl.ds(j * 16, 16)])
            return 0

        lax.fori_loop(0, CT, t_step, 0)
        pltpu.sync_copy(ob, out_hbm.at[pl.ds(n0 + q * CT, CT)])


@jax.jit
def kernel(x, Wg, W1, W2):
    W1b = W1.astype(jnp.bfloat16)
    x2 = x.reshape(N, D)
    wgp = jnp.pad(Wg, ((0, 0), (0, EP - E)))

    BR = 512
    i1, i2, w1r, w2r = pl.pallas_call(
        _router_body,
        grid=(N // BR,),
        in_specs=[
            pl.BlockSpec((BR, D), lambda i: (i, 0)),
            pl.BlockSpec((D, EP), lambda i: (0, 0)),
        ],
        out_specs=[
            pl.BlockSpec((BR,), lambda i: (i,)),
            pl.BlockSpec((BR,), lambda i: (i,)),
            pl.BlockSpec((BR,), lambda i: (i,)),
            pl.BlockSpec((BR,), lambda i: (i,)),
        ],
        out_shape=[
            jax.ShapeDtypeStruct((N,), jnp.int32),
            jax.ShapeDtypeStruct((N,), jnp.int32),
            jax.ShapeDtypeStruct((N,), jnp.float32),
            jax.ShapeDtypeStruct((N,), jnp.float32),
        ],
    )(x2, wgp)

    mesh = plsc.VectorSubcoreMesh(core_axis_name="c", subcore_axis_name="s")

    meta = pl.kernel(
        _meta_body,
        out_type=[
            jax.ShapeDtypeStruct((A,), jnp.int32),
            jax.ShapeDtypeStruct((48,), jnp.int32),
        ],
        mesh=mesh,
        compiler_params=pltpu.CompilerParams(needs_layout_passes=False),
        scratch_types=[
            pltpu.VMEM((A,), jnp.int32),
            pltpu.VMEM((CS,), jnp.int32),
            pltpu.VMEM((48,), jnp.int32),
            pltpu.SemaphoreType.DMA,
        ],
    )
    pos, te = meta(i1, i2)

    pos3 = pos.reshape(NC * NS, GQ, GR)
    xs = pl.kernel(
        _gather_body,
        out_type=jax.ShapeDtypeStruct((NROWS, D), jnp.float32),
        mesh=mesh,
        compiler_params=pltpu.CompilerParams(needs_layout_passes=False),
        scratch_types=[
            pltpu.VMEM((GQ, GR), jnp.int32),
            pltpu.VMEM((GR, D), jnp.float32),
            pltpu.VMEM((GR, D), jnp.float32),
            pltpu.SemaphoreType.DMA,
            pltpu.SemaphoreType.DMA,
        ],
    )(x2, pos3)

    grid_spec = pltpu.PrefetchScalarGridSpec(
        num_scalar_prefetch=1,
        grid=(NT,),
        in_specs=[
            pl.BlockSpec((BM, D), lambda m, te_r: (m, 0)),
            pl.BlockSpec(memory_space=pl.ANY),
            pl.BlockSpec(memory_space=pl.ANY),
        ],
        out_specs=pl.BlockSpec((BM, D), lambda m, te_r: (m, 0)),
        scratch_shapes=[
            pltpu.VMEM((2, D, DFF), jnp.bfloat16),
            pltpu.VMEM((2, DFF, D), jnp.float32),
            pltpu.SMEM((2,), jnp.int32),
            pltpu.SemaphoreType.DMA,
            pltpu.SemaphoreType.DMA,
            pltpu.SemaphoreType.DMA,
            pltpu.SemaphoreType.DMA,
        ],
    )
    ys = pl.pallas_call(
        _mlp_body,
        grid_spec=grid_spec,
        out_shape=jax.ShapeDtypeStruct((NROWS, D), jnp.float32),
        compiler_params=pltpu.CompilerParams(vmem_limit_bytes=100 * 1024 * 1024),
    )(te, xs, W1b, W2)

    posA = pos[:N].reshape(NC * NS, CQ, CT)
    posB = pos[N:].reshape(NC * NS, CQ, CT)
    out = pl.kernel(
        _combine_body,
        out_type=jax.ShapeDtypeStruct((N, D), jnp.float32),
        mesh=mesh,
        compiler_params=pltpu.CompilerParams(needs_layout_passes=False),
        scratch_types=[
            pltpu.VMEM((CQ, CT), jnp.int32),
            pltpu.VMEM((CQ, CT), jnp.int32),
            pltpu.VMEM((CQ * CT,), jnp.float32),
            pltpu.VMEM((CQ * CT,), jnp.float32),
            pltpu.VMEM((CT, D), jnp.float32),
            pltpu.VMEM((CT, D), jnp.float32),
            pltpu.VMEM((CT, D), jnp.float32),
            pltpu.VMEM((CT, D), jnp.float32),
            pltpu.VMEM((CT, D), jnp.float32),
            pltpu.SemaphoreType.DMA,
            pltpu.SemaphoreType.DMA,
            pltpu.SemaphoreType.DMA,
            pltpu.SemaphoreType.DMA,
        ],
    )(ys, posA, posB, w1r, w2r)
    return out.reshape(B, T, D)


# speedup vs baseline: 1.1278x; 1.0304x over previous
import math

import jax
import jax.numpy as jnp
from jax import lax
from jax.experimental import pallas as pl
from jax.experimental.pallas import tpu as pltpu
from jax.experimental.pallas import tpu_sc as plsc

B, T, D = 2, 2048, 1024
E = 8
DFF = 4 * D
N = B * T
A = 2 * N
BM = 256
NT = 40
NROWS = NT * BM
EP = 128
NEG = -1e30

NC, NS = 2, 16
CS = A // NS


def _router_body(x_ref, wg_ref, i1_ref, i2_ref, w1_ref, w2_ref):
    x = x_ref[...]
    logits = lax.dot_general(x, wg_ref[...], (((1,), (0,)), ((), ())),
                             preferred_element_type=jnp.float32)
    lane = lax.broadcasted_iota(jnp.int32, logits.shape, 1)
    valid = lane < E
    logits = jnp.where(valid, logits, NEG)
    m1 = jnp.max(logits, axis=1, keepdims=True)
    i1 = jnp.min(jnp.where(logits == m1, lane, EP), axis=1, keepdims=True)
    l2 = jnp.where(lane == i1, NEG, logits)
    m2 = jnp.max(l2, axis=1, keepdims=True)
    i2 = jnp.min(jnp.where(l2 == m2, lane, EP), axis=1, keepdims=True)
    e = jnp.exp(m2 - m1)
    w0 = 1.0 / (1.0 + e)
    i1_ref[...] = i1[:, 0]
    i2_ref[...] = i2[:, 0]
    w1_ref[...] = w0[:, 0]
    w2_ref[...] = (e * w0)[:, 0]


def _meta_body(i1_hbm, i2_hbm, pos_hbm, te_hbm, ids_v, pos_v, te_v, sem):
    c = lax.axis_index("c")
    s = lax.axis_index("s")
    lane = lax.iota(jnp.int32, 16)

    pltpu.sync_copy(i1_hbm, ids_v.at[pl.ds(0, N)])
    pltpu.sync_copy(i2_hbm, ids_v.at[pl.ds(N, N)])

    def h_step(v, hist):
        ev = ids_v[pl.ds(v * 16, 16)]
        for e in range(E):
            cnt = plsc.all_reduce_population_count(ev == e)
            hist = hist + jnp.where(lane == e, cnt, 0)
        return hist

    myv0 = s * (CS // 16)
    mybase = lax.fori_loop(0, myv0, h_step, jnp.zeros((16,), jnp.int32))
    g = lax.fori_loop(myv0, A // 16, h_step, mybase)
    ctiles = (g + (BM - 1)) // BM
    csz = ctiles * BM
    coff_incl = plsc.cumsum(csz)
    padded_off = coff_incl - csz
    base_vec = padded_off + mybase

    bases0 = tuple(jnp.sum(jnp.where(lane == e, base_vec, 0)) for e in range(E))

    def r_step(v, bases):
        ev = ids_v[pl.ds((myv0 + v) * 16, 16)]
        posv = jnp.zeros((16,), jnp.int32)
        new_bases = []
        for e in range(E):
            msk = ev == e
            r = plsc.cumsum(jnp.where(msk, 1, 0))
            posv = jnp.where(msk, bases[e] + r - 1, posv)
            new_bases.append(bases[e] + jnp.sum(jnp.where(msk, 1, 0)))
        pos_v[pl.ds(v * 16, 16)] = posv
        return tuple(new_bases)

    lax.fori_loop(0, CS // 16, r_step, bases0)

    wc = jnp.where(s >= 8, 1, 0)

    @pl.when(c == wc)
    def _():
        pltpu.sync_copy(pos_v, pos_hbm.at[pl.ds(s * CS, CS)])

    btile = plsc.cumsum(ctiles)
    bts = tuple(jnp.sum(jnp.where(lane == e, btile, 0)) for e in range(E))
    for vi in range(3):
        tvec = lax.iota(jnp.int32, 16) + vi * 16
        tev = jnp.zeros((16,), jnp.int32)
        for e in range(E):
            tev = tev + jnp.where(tvec >= bts[e], 1, 0)
        tev = jnp.minimum(tev, E - 1)
        if vi == 2:
            tev = jnp.where(tvec == 47, bts[E - 1], tev)
        te_v[pl.ds(vi * 16, 16)] = tev

    @pl.when((c == 0) & (s == 0))
    def _():
        pltpu.sync_copy(te_v, te_hbm)


GQ = 8
GR = 32


def _gather_body(x_hbm, pos_hbm, xs_hbm, idx_v, b0, b1, sem0, sem1):
    c = lax.axis_index("c")
    s = lax.axis_index("s")
    wid = s * NC + c
    row_base = lax.rem(wid, NS) * (GQ * GR)
    pltpu.sync_copy(pos_hbm.at[wid], idx_v)

    copies = []
    for q in range(GQ):
        b = b0 if q % 2 == 0 else b1
        sem = sem0 if q % 2 == 0 else sem1
        if q >= 2:
            copies[q - 2].wait()
        pltpu.sync_copy(x_hbm.at[pl.ds(row_base + q * GR, GR)], b)
        copies.append(pltpu.async_copy(b, xs_hbm.at[idx_v.at[q]], sem))
    copies[GQ - 2].wait()
    copies[GQ - 1].wait()


MLP_BK = 2048


def _mlp_body(te_ref, xs_ref, w1_hbm, w2_hbm, out_ref,
              w1b, w2b, ordr, s1a, s2a, s1b, s2b):
    m = pl.program_id(0)
    e = te_ref[m]
    sems = ((s1a, s2a), (s1b, s2b))

    def fetch(slot, eidx, sp):
        pltpu.make_async_copy(w1_hbm.at[eidx], w1b.at[slot], sp[0]).start()
        pltpu.make_async_copy(w2_hbm.at[eidx], w2b.at[slot], sp[1]).start()

    def wait_slot(slot, eidx, sp):
        pltpu.make_async_copy(w1_hbm.at[eidx], w1b.at[slot], sp[0]).wait()
        pltpu.make_async_copy(w2_hbm.at[eidx], w2b.at[slot], sp[1]).wait()

    nu = te_ref[47]
    active = m < nu
    prev_e = te_ref[jnp.maximum(m - 1, 0)]
    boundary = jnp.logical_and((m == 0) | (e != prev_e), active)

    @pl.when(m == 0)
    def _():
        ordr[0] = 0
        fetch(0, e, sems[0])

    @pl.when(boundary)
    def _():
        @pl.when(m > 0)
        def _():
            ordr[0] = ordr[0] + 1
        slot = lax.rem(ordr[0], 2)

        @pl.when(slot == 0)
        def _():
            wait_slot(0, e, sems[0])

        @pl.when(slot == 1)
        def _():
            wait_slot(1, e, sems[1])

        def scan(i, st):
            found, val = st
            cand = te_ref[i]
            take = jnp.logical_and(
                jnp.logical_not(found),
                jnp.logical_and(jnp.logical_and(i > m, i < nu), cand != e))
            return (jnp.logical_or(found, take), jnp.where(take, cand, val))

        _, en = lax.fori_loop(0, NT, scan, (False, jnp.int32(-1)))
        nslot = lax.rem(ordr[0] + 1, 2)

        @pl.when(jnp.logical_and(en >= 0, nslot == 0))
        def _():
            fetch(0, en, sems[0])

        @pl.when(jnp.logical_and(en >= 0, nslot == 1))
        def _():
            fetch(1, en, sems[1])

    @pl.when(active)
    def _():
        slot = lax.rem(ordr[0], 2)
        xb = xs_ref[...].astype(jnp.bfloat16)
        acc = jnp.zeros((BM, D), jnp.float32)
        for j in range(DFF // MLP_BK):
            w1j = w1b[slot, :, pl.ds(j * MLP_BK, MLP_BK)]
            hj = lax.dot_general(xb, w1j, (((1,), (0,)), ((), ())),
                                 preferred_element_type=jnp.float32)
            hj = 0.5 * hj * (1.0 + lax.erf(hj * (1.0 / math.sqrt(2.0))))
            w2j = w2b[slot, pl.ds(j * MLP_BK, MLP_BK), :].astype(jnp.bfloat16)
            acc = acc + lax.dot_general(hj.astype(jnp.bfloat16), w2j,
                                        (((1,), (0,)), ((), ())),
                                        preferred_element_type=jnp.float32)
        out_ref[...] = acc


CQ = 8
CT = 16


def _combine_body(ys_hbm, posA_hbm, posB_hbm, w1_hbm, w2_hbm, out_hbm,
                  idx0_v, idx1_v, w0_v, w1v_v, r0a, r1a, r0b, r1b, ob,
                  semg0, semg1, semg2, semg3):
    c = lax.axis_index("c")
    s = lax.axis_index("s")
    wid = s * NC + c
    n0 = wid * (CQ * CT)
    lane = lax.iota(jnp.int32, 16)
    pltpu.sync_copy(posA_hbm.at[wid], idx0_v)
    pltpu.sync_copy(posB_hbm.at[wid], idx1_v)
    pltpu.sync_copy(w1_hbm.at[pl.ds(n0, CQ * CT)], w0_v)
    pltpu.sync_copy(w2_hbm.at[pl.ds(n0, CQ * CT)], w1v_v)

    bufs = ((r0a, r1a, semg0, semg1), (r0b, r1b, semg2, semg3))

    def issue(q):
        b0, b1, sg0, sg1 = bufs[q % 2]
        cp0 = pltpu.async_copy(ys_hbm.at[idx0_v.at[q]], b0, sg0)
        cp1 = pltpu.async_copy(ys_hbm.at[idx1_v.at[q]], b1, sg1)
        return cp0, cp1

    cps = [issue(0)]
    for q in range(CQ):
        if q + 1 < CQ:
            cps.append(issue(q + 1))
        cps[q][0].wait()
        cps[q][1].wait()
        b0, b1 = bufs[q % 2][0], bufs[q % 2][1]
        wa = w0_v[pl.ds(q * 16, 16)]
        wb = w1v_v[pl.ds(q * 16, 16)]

        def t_step(t, _):
            w0s = jnp.sum(jnp.where(lane == t, wa, 0.0))
            w1s = jnp.sum(jnp.where(lane == t, wb, 0.0))
            for j in range(D // 16):
                ob[t, pl.ds(j * 16, 16)] = (
                    w0s * b0[t, pl.ds(j * 16, 16)]
                    + w1s * b1[t, pl.ds(j * 16, 16)])
            return 0

        lax.fori_loop(0, CT, t_step, 0)
        pltpu.sync_copy(ob, out_hbm.at[pl.ds(n0 + q * CT, CT)])


@jax.jit
def kernel(x, Wg, W1, W2):
    W1b = W1.astype(jnp.bfloat16)
    x2 = x.reshape(N, D)
    wgp = jnp.pad(Wg, ((0, 0), (0, EP - E)))

    BR = 4096
    i1, i2, w1r, w2r = pl.pallas_call(
        _router_body,
        grid=(N // BR,),
        in_specs=[
            pl.BlockSpec((BR, D), lambda i: (i, 0)),
            pl.BlockSpec((D, EP), lambda i: (0, 0)),
        ],
        out_specs=[
            pl.BlockSpec((BR,), lambda i: (i,)),
            pl.BlockSpec((BR,), lambda i: (i,)),
            pl.BlockSpec((BR,), lambda i: (i,)),
            pl.BlockSpec((BR,), lambda i: (i,)),
        ],
        out_shape=[
            jax.ShapeDtypeStruct((N,), jnp.int32),
            jax.ShapeDtypeStruct((N,), jnp.int32),
            jax.ShapeDtypeStruct((N,), jnp.float32),
            jax.ShapeDtypeStruct((N,), jnp.float32),
        ],
    )(x2, wgp)

    mesh = plsc.VectorSubcoreMesh(core_axis_name="c", subcore_axis_name="s")

    meta = pl.kernel(
        _meta_body,
        out_type=[
            jax.ShapeDtypeStruct((A,), jnp.int32),
            jax.ShapeDtypeStruct((48,), jnp.int32),
        ],
        mesh=mesh,
        compiler_params=pltpu.CompilerParams(needs_layout_passes=False),
        scratch_types=[
            pltpu.VMEM((A,), jnp.int32),
            pltpu.VMEM((CS,), jnp.int32),
            pltpu.VMEM((48,), jnp.int32),
            pltpu.SemaphoreType.DMA,
        ],
    )
    pos, te = meta(i1, i2)

    pos3 = pos.reshape(NC * NS, GQ, GR)
    xs = pl.kernel(
        _gather_body,
        out_type=jax.ShapeDtypeStruct((NROWS, D), jnp.float32),
        mesh=mesh,
        compiler_params=pltpu.CompilerParams(needs_layout_passes=False),
        scratch_types=[
            pltpu.VMEM((GQ, GR), jnp.int32),
            pltpu.VMEM((GR, D), jnp.float32),
            pltpu.VMEM((GR, D), jnp.float32),
            pltpu.SemaphoreType.DMA,
            pltpu.SemaphoreType.DMA,
        ],
    )(x2, pos3)

    grid_spec = pltpu.PrefetchScalarGridSpec(
        num_scalar_prefetch=1,
        grid=(NT,),
        in_specs=[
            pl.BlockSpec((BM, D), lambda m, te_r: (m, 0)),
            pl.BlockSpec(memory_space=pl.ANY),
            pl.BlockSpec(memory_space=pl.ANY),
        ],
        out_specs=pl.BlockSpec((BM, D), lambda m, te_r: (m, 0)),
        scratch_shapes=[
            pltpu.VMEM((2, D, DFF), jnp.bfloat16),
            pltpu.VMEM((2, DFF, D), jnp.float32),
            pltpu.SMEM((2,), jnp.int32),
            pltpu.SemaphoreType.DMA,
            pltpu.SemaphoreType.DMA,
            pltpu.SemaphoreType.DMA,
            pltpu.SemaphoreType.DMA,
        ],
    )
    ys = pl.pallas_call(
        _mlp_body,
        grid_spec=grid_spec,
        out_shape=jax.ShapeDtypeStruct((NROWS, D), jnp.float32),
        compiler_params=pltpu.CompilerParams(vmem_limit_bytes=100 * 1024 * 1024),
    )(te, xs, W1b, W2)

    posA = pos[:N].reshape(NC * NS, CQ, CT)
    posB = pos[N:].reshape(NC * NS, CQ, CT)
    out = pl.kernel(
        _combine_body,
        out_type=jax.ShapeDtypeStruct((N, D), jnp.float32),
        mesh=mesh,
        compiler_params=pltpu.CompilerParams(needs_layout_passes=False),
        scratch_types=[
            pltpu.VMEM((CQ, CT), jnp.int32),
            pltpu.VMEM((CQ, CT), jnp.int32),
            pltpu.VMEM((CQ * CT,), jnp.float32),
            pltpu.VMEM((CQ * CT,), jnp.float32),
            pltpu.VMEM((CT, D), jnp.float32),
            pltpu.VMEM((CT, D), jnp.float32),
            pltpu.VMEM((CT, D), jnp.float32),
            pltpu.VMEM((CT, D), jnp.float32),
            pltpu.VMEM((CT, D), jnp.float32),
            pltpu.SemaphoreType.DMA,
            pltpu.SemaphoreType.DMA,
            pltpu.SemaphoreType.DMA,
            pltpu.SemaphoreType.DMA,
        ],
    )(ys, posA, posB, w1r, w2r)
    return out.reshape(B, T, D)


# speedup vs baseline: 1.1335x; 1.0050x over previous
import math

import jax
import jax.numpy as jnp
from jax import lax
from jax.experimental import pallas as pl
from jax.experimental.pallas import tpu as pltpu
from jax.experimental.pallas import tpu_sc as plsc

B, T, D = 2, 2048, 1024
E = 8
DFF = 4 * D
N = B * T
A = 2 * N
BM = 256
NT = 40
NROWS = NT * BM
EP = 128
NEG = -1e30

NC, NS = 2, 16
CS = A // NS


def _router_body(x_ref, wg_ref, i1_ref, i2_ref, w1_ref, w2_ref):
    x = x_ref[...]
    logits = lax.dot_general(x, wg_ref[...], (((1,), (0,)), ((), ())),
                             preferred_element_type=jnp.float32)
    lane = lax.broadcasted_iota(jnp.int32, logits.shape, 1)
    valid = lane < E
    logits = jnp.where(valid, logits, NEG)
    m1 = jnp.max(logits, axis=1, keepdims=True)
    i1 = jnp.min(jnp.where(logits == m1, lane, EP), axis=1, keepdims=True)
    l2 = jnp.where(lane == i1, NEG, logits)
    m2 = jnp.max(l2, axis=1, keepdims=True)
    i2 = jnp.min(jnp.where(l2 == m2, lane, EP), axis=1, keepdims=True)
    e = jnp.exp(m2 - m1)
    w0 = 1.0 / (1.0 + e)
    i1_ref[...] = i1[:, 0]
    i2_ref[...] = i2[:, 0]
    w1_ref[...] = w0[:, 0]
    w2_ref[...] = (e * w0)[:, 0]


def _meta_body(i1_hbm, i2_hbm, pos_hbm, te_hbm, ids_v, pos_v, te_v, sem):
    c = lax.axis_index("c")
    s = lax.axis_index("s")
    lane = lax.iota(jnp.int32, 16)

    pltpu.sync_copy(i1_hbm, ids_v.at[pl.ds(0, N)])
    pltpu.sync_copy(i2_hbm, ids_v.at[pl.ds(N, N)])

    def h_step(v, hist):
        ev = ids_v[pl.ds(v * 16, 16)]
        for e in range(E):
            cnt = plsc.all_reduce_population_count(ev == e)
            hist = hist + jnp.where(lane == e, cnt, 0)
        return hist

    myv0 = s * (CS // 16)
    mybase = lax.fori_loop(0, myv0, h_step, jnp.zeros((16,), jnp.int32))
    g = lax.fori_loop(myv0, A // 16, h_step, mybase)
    ctiles = (g + (BM - 1)) // BM
    csz = ctiles * BM
    coff_incl = plsc.cumsum(csz)
    padded_off = coff_incl - csz
    base_vec = padded_off + mybase

    bases0 = tuple(jnp.sum(jnp.where(lane == e, base_vec, 0)) for e in range(E))

    def r_step(v, bases):
        ev = ids_v[pl.ds((myv0 + v) * 16, 16)]
        posv = jnp.zeros((16,), jnp.int32)
        new_bases = []
        for e in range(E):
            msk = ev == e
            r = plsc.cumsum(jnp.where(msk, 1, 0))
            posv = jnp.where(msk, bases[e] + r - 1, posv)
            new_bases.append(bases[e] + jnp.sum(jnp.where(msk, 1, 0)))
        pos_v[pl.ds(v * 16, 16)] = posv
        return tuple(new_bases)

    lax.fori_loop(0, CS // 16, r_step, bases0)

    wc = jnp.where(s >= 8, 1, 0)

    @pl.when(c == wc)
    def _():
        pltpu.sync_copy(pos_v, pos_hbm.at[pl.ds(s * CS, CS)])

    btile = plsc.cumsum(ctiles)
    bts = tuple(jnp.sum(jnp.where(lane == e, btile, 0)) for e in range(E))
    for vi in range(3):
        tvec = lax.iota(jnp.int32, 16) + vi * 16
        tev = jnp.zeros((16,), jnp.int32)
        for e in range(E):
            tev = tev + jnp.where(tvec >= bts[e], 1, 0)
        tev = jnp.minimum(tev, E - 1)
        if vi == 2:
            tev = jnp.where(tvec == 47, bts[E - 1], tev)
        te_v[pl.ds(vi * 16, 16)] = tev

    @pl.when((c == 0) & (s == 0))
    def _():
        pltpu.sync_copy(te_v, te_hbm)


GQ = 8
GR = 32


def _gather_body(x_hbm, pos_hbm, xs_hbm, idx_v, b0, b1, sem0, sem1):
    c = lax.axis_index("c")
    s = lax.axis_index("s")
    wid = s * NC + c
    row_base = lax.rem(wid, NS) * (GQ * GR)
    pltpu.sync_copy(pos_hbm.at[wid], idx_v)

    copies = []
    for q in range(GQ):
        b = b0 if q % 2 == 0 else b1
        sem = sem0 if q % 2 == 0 else sem1
        if q >= 2:
            copies[q - 2].wait()
        pltpu.sync_copy(x_hbm.at[pl.ds(row_base + q * GR, GR)], b)
        copies.append(pltpu.async_copy(b, xs_hbm.at[idx_v.at[q]], sem))
    copies[GQ - 2].wait()
    copies[GQ - 1].wait()


MLP_BK = 2048


def _mlp_body(te_ref, xs_ref, w1_hbm, w2_hbm, out_ref,
              w1b, w2b, ordr, s1a, s2a, s1b, s2b):
    m = pl.program_id(0)
    e = te_ref[m]
    sems = ((s1a, s2a), (s1b, s2b))

    def fetch(slot, eidx, sp):
        pltpu.make_async_copy(w1_hbm.at[eidx], w1b.at[slot], sp[0]).start()
        pltpu.make_async_copy(w2_hbm.at[eidx], w2b.at[slot], sp[1]).start()

    def wait_slot(slot, eidx, sp):
        pltpu.make_async_copy(w1_hbm.at[eidx], w1b.at[slot], sp[0]).wait()
        pltpu.make_async_copy(w2_hbm.at[eidx], w2b.at[slot], sp[1]).wait()

    nu = te_ref[47]
    active = m < nu
    prev_e = te_ref[jnp.maximum(m - 1, 0)]
    boundary = jnp.logical_and((m == 0) | (e != prev_e), active)

    @pl.when(m == 0)
    def _():
        ordr[0] = 0
        fetch(0, e, sems[0])

    @pl.when(boundary)
    def _():
        @pl.when(m > 0)
        def _():
            ordr[0] = ordr[0] + 1
        slot = lax.rem(ordr[0], 2)

        @pl.when(slot == 0)
        def _():
            wait_slot(0, e, sems[0])

        @pl.when(slot == 1)
        def _():
            wait_slot(1, e, sems[1])

        def scan(i, st):
            found, val = st
            cand = te_ref[i]
            take = jnp.logical_and(
                jnp.logical_not(found),
                jnp.logical_and(jnp.logical_and(i > m, i < nu), cand != e))
            return (jnp.logical_or(found, take), jnp.where(take, cand, val))

        _, en = lax.fori_loop(0, NT, scan, (False, jnp.int32(-1)))
        nslot = lax.rem(ordr[0] + 1, 2)

        @pl.when(jnp.logical_and(en >= 0, nslot == 0))
        def _():
            fetch(0, en, sems[0])

        @pl.when(jnp.logical_and(en >= 0, nslot == 1))
        def _():
            fetch(1, en, sems[1])

    @pl.when(active)
    def _():
        slot = lax.rem(ordr[0], 2)
        xb = xs_ref[...].astype(jnp.bfloat16)
        acc = jnp.zeros((BM, D), jnp.float32)
        for j in range(DFF // MLP_BK):
            w1j = w1b[slot, :, pl.ds(j * MLP_BK, MLP_BK)]
            hj = lax.dot_general(xb, w1j, (((1,), (0,)), ((), ())),
                                 preferred_element_type=jnp.float32)
            hj = 0.5 * hj * (1.0 + lax.erf(hj * (1.0 / math.sqrt(2.0))))
            w2j = w2b[slot, pl.ds(j * MLP_BK, MLP_BK), :].astype(jnp.bfloat16)
            acc = acc + lax.dot_general(hj.astype(jnp.bfloat16), w2j,
                                        (((1,), (0,)), ((), ())),
                                        preferred_element_type=jnp.float32)
        out_ref[...] = acc


CQ = 8
CT = 16


def _combine_body(ys_hbm, posA_hbm, posB_hbm, w1_hbm, w2_hbm, out_hbm,
                  idx0_v, idx1_v, w0_v, w1v_v, r0a, r1a, r0b, r1b, ob,
                  semg0, semg1, semg2, semg3):
    c = lax.axis_index("c")
    s = lax.axis_index("s")
    wid = s * NC + c
    n0 = wid * (CQ * CT)
    lane = lax.iota(jnp.int32, 16)
    pltpu.sync_copy(posA_hbm.at[wid], idx0_v)
    pltpu.sync_copy(posB_hbm.at[wid], idx1_v)
    pltpu.sync_copy(w1_hbm.at[pl.ds(n0, CQ * CT)], w0_v)
    pltpu.sync_copy(w2_hbm.at[pl.ds(n0, CQ * CT)], w1v_v)

    bufs = ((r0a, r1a, semg0, semg1), (r0b, r1b, semg2, semg3))

    def issue(q):
        b0, b1, sg0, sg1 = bufs[q % 2]
        cp0 = pltpu.async_copy(ys_hbm.at[idx0_v.at[q]], b0, sg0)
        cp1 = pltpu.async_copy(ys_hbm.at[idx1_v.at[q]], b1, sg1)
        return cp0, cp1

    cps = [issue(0)]
    for q in range(CQ):
        if q + 1 < CQ:
            cps.append(issue(q + 1))
        cps[q][0].wait()
        cps[q][1].wait()
        b0, b1 = bufs[q % 2][0], bufs[q % 2][1]
        wa = w0_v[pl.ds(q * 16, 16)]
        wb = w1v_v[pl.ds(q * 16, 16)]

        def t_step(t, _):
            w0s = jnp.sum(jnp.where(lane == t, wa, 0.0))
            w1s = jnp.sum(jnp.where(lane == t, wb, 0.0))
            for j in range(D // 16):
                ob[t, pl.ds(j * 16, 16)] = (
                    w0s * b0[t, pl.ds(j * 16, 16)]
                    + w1s * b1[t, pl.ds(j * 16, 16)])
            return 0

        lax.fori_loop(0, CT, t_step, 0)
        pltpu.sync_copy(ob, out_hbm.at[pl.ds(n0 + q * CT, CT)])


@jax.jit
def kernel(x, Wg, W1, W2):
    W1b = W1.astype(jnp.bfloat16)
    x2 = x.reshape(N, D)
    wgp = jnp.pad(Wg, ((0, 0), (0, EP - E)))

    BR = 2048
    i1, i2, w1r, w2r = pl.pallas_call(
        _router_body,
        grid=(N // BR,),
        in_specs=[
            pl.BlockSpec((BR, D), lambda i: (i, 0)),
            pl.BlockSpec((D, EP), lambda i: (0, 0)),
        ],
        out_specs=[
            pl.BlockSpec((BR,), lambda i: (i,)),
            pl.BlockSpec((BR,), lambda i: (i,)),
            pl.BlockSpec((BR,), lambda i: (i,)),
            pl.BlockSpec((BR,), lambda i: (i,)),
        ],
        out_shape=[
            jax.ShapeDtypeStruct((N,), jnp.int32),
            jax.ShapeDtypeStruct((N,), jnp.int32),
            jax.ShapeDtypeStruct((N,), jnp.float32),
            jax.ShapeDtypeStruct((N,), jnp.float32),
        ],
    )(x2, wgp)

    mesh = plsc.VectorSubcoreMesh(core_axis_name="c", subcore_axis_name="s")

    meta = pl.kernel(
        _meta_body,
        out_type=[
            jax.ShapeDtypeStruct((A,), jnp.int32),
            jax.ShapeDtypeStruct((48,), jnp.int32),
        ],
        mesh=mesh,
        compiler_params=pltpu.CompilerParams(needs_layout_passes=False),
        scratch_types=[
            pltpu.VMEM((A,), jnp.int32),
            pltpu.VMEM((CS,), jnp.int32),
            pltpu.VMEM((48,), jnp.int32),
            pltpu.SemaphoreType.DMA,
        ],
    )
    pos, te = meta(i1, i2)

    pos3 = pos.reshape(NC * NS, GQ, GR)
    xs = pl.kernel(
        _gather_body,
        out_type=jax.ShapeDtypeStruct((NROWS, D), jnp.float32),
        mesh=mesh,
        compiler_params=pltpu.CompilerParams(needs_layout_passes=False),
        scratch_types=[
            pltpu.VMEM((GQ, GR), jnp.int32),
            pltpu.VMEM((GR, D), jnp.float32),
            pltpu.VMEM((GR, D), jnp.float32),
            pltpu.SemaphoreType.DMA,
            pltpu.SemaphoreType.DMA,
        ],
    )(x2, pos3)

    grid_spec = pltpu.PrefetchScalarGridSpec(
        num_scalar_prefetch=1,
        grid=(NT,),
        in_specs=[
            pl.BlockSpec((BM, D), lambda m, te_r: (m, 0)),
            pl.BlockSpec(memory_space=pl.ANY),
            pl.BlockSpec(memory_space=pl.ANY),
        ],
        out_specs=pl.BlockSpec((BM, D), lambda m, te_r: (m, 0)),
        scratch_shapes=[
            pltpu.VMEM((2, D, DFF), jnp.bfloat16),
            pltpu.VMEM((2, DFF, D), jnp.float32),
            pltpu.SMEM((2,), jnp.int32),
            pltpu.SemaphoreType.DMA,
            pltpu.SemaphoreType.DMA,
            pltpu.SemaphoreType.DMA,
            pltpu.SemaphoreType.DMA,
        ],
    )
    ys = pl.pallas_call(
        _mlp_body,
        grid_spec=grid_spec,
        out_shape=jax.ShapeDtypeStruct((NROWS, D), jnp.float32),
        compiler_params=pltpu.CompilerParams(vmem_limit_bytes=100 * 1024 * 1024),
    )(te, xs, W1b, W2)

    posA = pos[:N].reshape(NC * NS, CQ, CT)
    posB = pos[N:].reshape(NC * NS, CQ, CT)
    out = pl.kernel(
        _combine_body,
        out_type=jax.ShapeDtypeStruct((N, D), jnp.float32),
        mesh=mesh,
        compiler_params=pltpu.CompilerParams(needs_layout_passes=False),
        scratch_types=[
            pltpu.VMEM((CQ, CT), jnp.int32),
            pltpu.VMEM((CQ, CT), jnp.int32),
            pltpu.VMEM((CQ * CT,), jnp.float32),
            pltpu.VMEM((CQ * CT,), jnp.float32),
            pltpu.VMEM((CT, D), jnp.float32),
            pltpu.VMEM((CT, D), jnp.float32),
            pltpu.VMEM((CT, D), jnp.float32),
            pltpu.VMEM((CT, D), jnp.float32),
            pltpu.VMEM((CT, D), jnp.float32),
            pltpu.SemaphoreType.DMA,
            pltpu.SemaphoreType.DMA,
            pltpu.SemaphoreType.DMA,
            pltpu.SemaphoreType.DMA,
        ],
    )(ys, posA, posB, w1r, w2r)
    return out.reshape(B, T, D)
